# baseline (speedup 1.0000x reference)
import jax
import jax.numpy as jnp
from jax import lax
from jax.experimental import pallas as pl
from jax.experimental.pallas import tpu as pltpu
from jax.experimental.pallas import tpu_sc as plsc

MU = 3.5
B = 16384
F = 32
NC, NS, L = 2, 16, 16
NW = NC * NS
RPW = B // NW
CHUNK = 128
NCHUNK = RPW // CHUNK
HPAD = 17


def _svdpp_body(user_hbm, item_hbm, P_hbm, Q_hbm, bu_hbm, bi_hbm, out_hbm,
                uidx, iidx, pu, qi, buv, biv, hbuf, outv, sem):
    wid = lax.axis_index("s") * NC + lax.axis_index("c")
    base = wid * RPW

    pltpu.sync_copy(user_hbm.at[pl.ds(wid * NCHUNK, NCHUNK)], uidx)
    pltpu.sync_copy(item_hbm.at[pl.ds(wid * NCHUNK, NCHUNK)], iidx)

    copies = []
    for j in range(NCHUNK):
        sl = pl.ds(j * CHUNK, CHUNK)
        copies.append(pltpu.async_copy(P_hbm.at[uidx.at[j]], pu.at[sl], sem))
        copies.append(pltpu.async_copy(Q_hbm.at[iidx.at[j]], qi.at[sl], sem))
        copies.append(pltpu.async_copy(bu_hbm.at[uidx.at[j]], buv.at[sl], sem))
        copies.append(pltpu.async_copy(bi_hbm.at[iidx.at[j]], biv.at[sl], sem))
    for c in copies:
        c.wait()

    lanes = lax.iota(jnp.int32, L)

    def s1(b, carry):
        p = pu[b, :]
        q = qi[b, :]
        p0, p1 = plsc.unpack(p, format=plsc.PackFormat.INTERLEAVED)
        q0, q1 = plsc.unpack(q, format=plsc.PackFormat.INTERLEAVED)
        plsc.store_scatter(hbuf, [b * HPAD + lanes], p0 * q0 + p1 * q1)
        return carry

    lax.fori_loop(0, RPW, s1, 0)

    def s2(i, carry):
        rvec = i * L + lanes
        acc = plsc.load_gather(buv, [rvec]) + plsc.load_gather(biv, [rvec]) + MU
        hbase = rvec * HPAD
        for j in range(L):
            acc = acc + plsc.load_gather(hbuf, [hbase + j])
        plsc.store_scatter(outv, [rvec], acc)
        return carry

    lax.fori_loop(0, RPW // L, s2, 0)

    pltpu.sync_copy(outv, out_hbm.at[pl.ds(base, RPW)])


def kernel(x, P, Q, bu, bi):
    user = x[:, 0].astype(jnp.int32).reshape(B // CHUNK, CHUNK)
    item = x[:, 1].astype(jnp.int32).reshape(B // CHUNK, CHUNK)
    Pb = P.astype(jnp.bfloat16)
    Qb = Q.astype(jnp.bfloat16)
    mesh = plsc.VectorSubcoreMesh(core_axis_name="c", subcore_axis_name="s")
    k = pl.kernel(
        _svdpp_body,
        out_type=jax.ShapeDtypeStruct((B,), jnp.float32),
        mesh=mesh,
        compiler_params=pltpu.CompilerParams(
            needs_layout_passes=False, use_tc_tiling_on_sc=False),
        scratch_types=[
            pltpu.VMEM((NCHUNK, CHUNK), jnp.int32),
            pltpu.VMEM((NCHUNK, CHUNK), jnp.int32),
            pltpu.VMEM((RPW, F), jnp.bfloat16),
            pltpu.VMEM((RPW, F), jnp.bfloat16),
            pltpu.VMEM((RPW,), jnp.float32),
            pltpu.VMEM((RPW,), jnp.float32),
            pltpu.VMEM((RPW * HPAD,), jnp.float32),
            pltpu.VMEM((RPW,), jnp.float32),
            pltpu.SemaphoreType.DMA,
        ],
    )
    return k(user, item, Pb, Qb, bu[:, 0], bi[:, 0])


# speedup vs baseline: 1.5007x; 1.5007x over previous
import jax
import jax.numpy as jnp
from jax import lax
from jax.experimental import pallas as pl
from jax.experimental.pallas import tpu as pltpu
from jax.experimental.pallas import tpu_sc as plsc

MU = 3.5
B = 16384
F = 32
N = 1000000
NC, NS, L = 2, 16, 16
NW = NC * NS
RPW = B // NW
CHUNK = 128
NCHUNK = RPW // CHUNK
HPAD = 17

TBLK = 2048
SLAB = TBLK // 4
TGRID = (N + TBLK - 1) // TBLK
NPAD = TGRID * TBLK


def _tpose_body(pt_ref, qt_ref, po_ref, qo_ref):
    for src, dst in ((pt_ref, po_ref), (qt_ref, qo_ref)):
        z = src[...].T
        y = jnp.concatenate(
            [z[a * SLAB:(a + 1) * SLAB] for a in range(4)], axis=1)
        dst[...] = y.reshape(TBLK * F)


def _svdpp_body(ut_hbm, it_hbm, uo_hbm, io_hbm, P_hbm, Q_hbm,
                bu_hbm, bi_hbm, out_hbm,
                uidx, iidx, uorig, iorig, pu, qi, buv, biv, hbuf, outv, sem):
    wid = lax.axis_index("s") * NC + lax.axis_index("c")
    base = wid * RPW

    pltpu.sync_copy(ut_hbm.at[pl.ds(wid * NCHUNK, NCHUNK)], uidx)
    pltpu.sync_copy(it_hbm.at[pl.ds(wid * NCHUNK, NCHUNK)], iidx)
    pltpu.sync_copy(uo_hbm.at[pl.ds(wid * NCHUNK, NCHUNK)], uorig)
    pltpu.sync_copy(io_hbm.at[pl.ds(wid * NCHUNK, NCHUNK)], iorig)

    copies = []
    for j in range(NCHUNK):
        sl = pl.ds(j * CHUNK, CHUNK)
        copies.append(pltpu.async_copy(P_hbm.at[uidx.at[j]], pu.at[sl], sem))
        copies.append(pltpu.async_copy(Q_hbm.at[iidx.at[j]], qi.at[sl], sem))
        copies.append(pltpu.async_copy(bu_hbm.at[uorig.at[j]], buv.at[sl], sem))
        copies.append(pltpu.async_copy(bi_hbm.at[iorig.at[j]], biv.at[sl], sem))
    for c in copies:
        c.wait()

    lanes = lax.iota(jnp.int32, L)

    def s1(b, carry):
        p0 = pu[b, pl.ds(0, L)]
        p1 = pu[b, pl.ds(L, L)]
        q0 = qi[b, pl.ds(0, L)]
        q1 = qi[b, pl.ds(L, L)]
        plsc.store_scatter(hbuf, [b * HPAD + lanes], p0 * q0 + p1 * q1)
        return carry

    lax.fori_loop(0, RPW, s1, 0)

    def s2(i, carry):
        rvec = i * L + lanes
        acc = plsc.load_gather(buv, [rvec]) + plsc.load_gather(biv, [rvec]) + MU
        hbase = rvec * HPAD
        for j in range(L):
            acc = acc + plsc.load_gather(hbuf, [hbase + j])
        plsc.store_scatter(outv, [rvec], acc)
        return carry

    lax.fori_loop(0, RPW // L, s2, 0)

    pltpu.sync_copy(outv, out_hbm.at[pl.ds(base, RPW)])


def kernel(x, P, Q, bu, bi):
    user = x[:, 0].astype(jnp.int32)
    item = x[:, 1].astype(jnp.int32)

    def swz(r):
        q = r % TBLK
        return (r // TBLK) * TBLK + 4 * (q % SLAB) + q // SLAB

    userT = swz(user).reshape(B // CHUNK, CHUNK)
    itemT = swz(item).reshape(B // CHUNK, CHUNK)
    userO = user.reshape(B // CHUNK, CHUNK)
    itemO = item.reshape(B // CHUNK, CHUNK)

    pb_flat, qb_flat = pl.pallas_call(
        _tpose_body,
        grid=(TGRID,),
        in_specs=[
            pl.BlockSpec((F, TBLK), lambda i: (0, i)),
            pl.BlockSpec((F, TBLK), lambda i: (0, i)),
        ],
        out_specs=[
            pl.BlockSpec((TBLK * F,), lambda i: (i,)),
            pl.BlockSpec((TBLK * F,), lambda i: (i,)),
        ],
        out_shape=[
            jax.ShapeDtypeStruct((NPAD * F,), jnp.float32),
            jax.ShapeDtypeStruct((NPAD * F,), jnp.float32),
        ],
    )(P.T, Q.T)
    Pb = pb_flat.reshape(NPAD, F)
    Qb = qb_flat.reshape(NPAD, F)

    mesh = plsc.VectorSubcoreMesh(core_axis_name="c", subcore_axis_name="s")
    k = pl.kernel(
        _svdpp_body,
        out_type=jax.ShapeDtypeStruct((B,), jnp.float32),
        mesh=mesh,
        compiler_params=pltpu.CompilerParams(
            needs_layout_passes=False, use_tc_tiling_on_sc=False),
        scratch_types=[
            pltpu.VMEM((NCHUNK, CHUNK), jnp.int32),
            pltpu.VMEM((NCHUNK, CHUNK), jnp.int32),
            pltpu.VMEM((NCHUNK, CHUNK), jnp.int32),
            pltpu.VMEM((NCHUNK, CHUNK), jnp.int32),
            pltpu.VMEM((RPW, F), jnp.float32),
            pltpu.VMEM((RPW, F), jnp.float32),
            pltpu.VMEM((RPW,), jnp.float32),
            pltpu.VMEM((RPW,), jnp.float32),
            pltpu.VMEM((RPW * HPAD,), jnp.float32),
            pltpu.VMEM((RPW,), jnp.float32),
            pltpu.SemaphoreType.DMA,
        ],
    )
    return k(userT, itemT, userO, itemO, Pb, Qb, bu[:, 0], bi[:, 0])


# speedup vs baseline: 2.1618x; 1.4406x over previous
import jax
import jax.numpy as jnp
from jax import lax
from jax.experimental import pallas as pl
from jax.experimental.pallas import tpu as pltpu
from jax.experimental.pallas import tpu_sc as plsc

MU = 3.5
B = 16384
F = 32
N = 1000000
NC, NS, L = 2, 16, 16
NW = NC * NS
RPW = B // NW
CHUNK = 128
NCHUNK = RPW // CHUNK
HPAD = 17

TBLK = 4096
SLAB = TBLK // 4
TGRID = (N + TBLK - 1) // TBLK
NPAD = TGRID * TBLK


def _tpose_body(pt_ref, qt_ref, po_ref, qo_ref):
    kk = lax.broadcasted_iota(jnp.int32, (F, 128), 0)
    jj = lax.broadcasted_iota(jnp.int32, (F, 128), 1)
    dn = (((0,), (0,)), ((), ()))
    for src, dst in ((pt_ref, po_ref), (qt_ref, qo_ref)):
        x = src[...]
        y = None
        for a in range(4):
            sel = (jj == kk + 32 * a).astype(jnp.float32)
            part = lax.dot_general(
                x[:, a * SLAB:(a + 1) * SLAB], sel, dn,
                preferred_element_type=jnp.float32)
            y = part if y is None else y + part
        dst[...] = y.reshape(TBLK * F)


def _svdpp_body(ut_hbm, it_hbm, uo_hbm, io_hbm, P_hbm, Q_hbm,
                bu_hbm, bi_hbm, out_hbm,
                uidx, iidx, uorig, iorig, pu, qi, buv, biv, hbuf, outv, sem):
    wid = lax.axis_index("s") * NC + lax.axis_index("c")
    base = wid * RPW

    pltpu.sync_copy(ut_hbm.at[pl.ds(wid * NCHUNK, NCHUNK)], uidx)
    pltpu.sync_copy(it_hbm.at[pl.ds(wid * NCHUNK, NCHUNK)], iidx)
    pltpu.sync_copy(uo_hbm.at[pl.ds(wid * NCHUNK, NCHUNK)], uorig)
    pltpu.sync_copy(io_hbm.at[pl.ds(wid * NCHUNK, NCHUNK)], iorig)

    copies = []
    for j in range(NCHUNK):
        sl = pl.ds(j * CHUNK, CHUNK)
        copies.append(pltpu.async_copy(P_hbm.at[uidx.at[j]], pu.at[sl], sem))
        copies.append(pltpu.async_copy(Q_hbm.at[iidx.at[j]], qi.at[sl], sem))
        copies.append(pltpu.async_copy(bu_hbm.at[uorig.at[j]], buv.at[sl], sem))
        copies.append(pltpu.async_copy(bi_hbm.at[iorig.at[j]], biv.at[sl], sem))
    for c in copies:
        c.wait()

    lanes = lax.iota(jnp.int32, L)

    def s1(b, carry):
        p0 = pu[b, pl.ds(0, L)]
        p1 = pu[b, pl.ds(L, L)]
        q0 = qi[b, pl.ds(0, L)]
        q1 = qi[b, pl.ds(L, L)]
        plsc.store_scatter(hbuf, [b * HPAD + lanes], p0 * q0 + p1 * q1)
        return carry

    lax.fori_loop(0, RPW, s1, 0)

    def s2(i, carry):
        rvec = i * L + lanes
        acc = plsc.load_gather(buv, [rvec]) + plsc.load_gather(biv, [rvec]) + MU
        hbase = rvec * HPAD
        for j in range(L):
            acc = acc + plsc.load_gather(hbuf, [hbase + j])
        plsc.store_scatter(outv, [rvec], acc)
        return carry

    lax.fori_loop(0, RPW // L, s2, 0)

    pltpu.sync_copy(outv, out_hbm.at[pl.ds(base, RPW)])


def kernel(x, P, Q, bu, bi):
    user = x[:, 0].astype(jnp.int32)
    item = x[:, 1].astype(jnp.int32)

    def swz(r):
        q = r % TBLK
        return (r // TBLK) * TBLK + 4 * (q % SLAB) + q // SLAB

    userT = swz(user).reshape(B // CHUNK, CHUNK)
    itemT = swz(item).reshape(B // CHUNK, CHUNK)
    userO = user.reshape(B // CHUNK, CHUNK)
    itemO = item.reshape(B // CHUNK, CHUNK)

    pb_flat, qb_flat = pl.pallas_call(
        _tpose_body,
        grid=(TGRID,),
        in_specs=[
            pl.BlockSpec((F, TBLK), lambda i: (0, i)),
            pl.BlockSpec((F, TBLK), lambda i: (0, i)),
        ],
        out_specs=[
            pl.BlockSpec((TBLK * F,), lambda i: (i,)),
            pl.BlockSpec((TBLK * F,), lambda i: (i,)),
        ],
        out_shape=[
            jax.ShapeDtypeStruct((NPAD * F,), jnp.float32),
            jax.ShapeDtypeStruct((NPAD * F,), jnp.float32),
        ],
        compiler_params=pltpu.CompilerParams(
            fuse_transposed_lhs_in_matmul=True),
    )(P.T, Q.T)
    Pb = pb_flat.reshape(NPAD, F)
    Qb = qb_flat.reshape(NPAD, F)

    mesh = plsc.VectorSubcoreMesh(core_axis_name="c", subcore_axis_name="s")
    k = pl.kernel(
        _svdpp_body,
        out_type=jax.ShapeDtypeStruct((B,), jnp.float32),
        mesh=mesh,
        compiler_params=pltpu.CompilerParams(
            needs_layout_passes=False, use_tc_tiling_on_sc=False),
        scratch_types=[
            pltpu.VMEM((NCHUNK, CHUNK), jnp.int32),
            pltpu.VMEM((NCHUNK, CHUNK), jnp.int32),
            pltpu.VMEM((NCHUNK, CHUNK), jnp.int32),
            pltpu.VMEM((NCHUNK, CHUNK), jnp.int32),
            pltpu.VMEM((RPW, F), jnp.float32),
            pltpu.VMEM((RPW, F), jnp.float32),
            pltpu.VMEM((RPW,), jnp.float32),
            pltpu.VMEM((RPW,), jnp.float32),
            pltpu.VMEM((RPW * HPAD,), jnp.float32),
            pltpu.VMEM((RPW,), jnp.float32),
            pltpu.SemaphoreType.DMA,
        ],
    )
    return k(userT, itemT, userO, itemO, Pb, Qb, bu[:, 0], bi[:, 0])


# speedup vs baseline: 3.0634x; 1.4171x over previous
import jax
import jax.numpy as jnp
from jax import lax
from jax.experimental import pallas as pl
from jax.experimental.pallas import tpu as pltpu
from jax.experimental.pallas import tpu_sc as plsc

MU = 3.5
B = 16384
F = 32
N = 1000000
NC, NS, L = 2, 16, 16
NW = NC * NS
RPW = B // NW
CHUNK = 128
NCHUNK = RPW // CHUNK
HPAD = 17

TBLK = 4096
SLAB = TBLK // 4
TGRID = (N + TBLK - 1) // TBLK
NPAD = TGRID * TBLK


def _tpose_body(pt_ref, qt_ref, but_ref, bit_ref, po_ref, qo_ref,
                buo_ref, bio_ref):
    kk = lax.broadcasted_iota(jnp.int32, (F, 128), 0)
    jj = lax.broadcasted_iota(jnp.int32, (F, 128), 1)
    dn = (((0,), (0,)), ((), ()))
    for src, dst in ((pt_ref, po_ref), (qt_ref, qo_ref)):
        x = src[...].astype(jnp.bfloat16)
        y = None
        for a in range(4):
            sel = (jj == kk + 32 * a).astype(jnp.bfloat16)
            part = lax.dot_general(
                x[:, a * SLAB:(a + 1) * SLAB], sel, dn,
                preferred_element_type=jnp.float32)
            y = part if y is None else y + part
        dst[...] = y.reshape(TBLK * F)
    buo_ref[...] = but_ref[...].reshape(TBLK)
    bio_ref[...] = bit_ref[...].reshape(TBLK)


def _svdpp_body(ut_hbm, it_hbm, uo_hbm, io_hbm, P_hbm, Q_hbm,
                bu_hbm, bi_hbm, out_hbm,
                uidx, iidx, uorig, iorig, pu, qi, buv, biv, hbuf, outv, sem):
    wid = lax.axis_index("s") * NC + lax.axis_index("c")
    base = wid * RPW

    pltpu.sync_copy(ut_hbm.at[pl.ds(wid * NCHUNK, NCHUNK)], uidx)
    pltpu.sync_copy(it_hbm.at[pl.ds(wid * NCHUNK, NCHUNK)], iidx)
    pltpu.sync_copy(uo_hbm.at[pl.ds(wid * NCHUNK, NCHUNK)], uorig)
    pltpu.sync_copy(io_hbm.at[pl.ds(wid * NCHUNK, NCHUNK)], iorig)

    copies = []
    for j in range(NCHUNK):
        sl = pl.ds(j * CHUNK, CHUNK)
        copies.append(pltpu.async_copy(P_hbm.at[uidx.at[j]], pu.at[sl], sem))
        copies.append(pltpu.async_copy(Q_hbm.at[iidx.at[j]], qi.at[sl], sem))
        copies.append(pltpu.async_copy(bu_hbm.at[uorig.at[j]], buv.at[sl], sem))
        copies.append(pltpu.async_copy(bi_hbm.at[iorig.at[j]], biv.at[sl], sem))
    for c in copies:
        c.wait()

    lanes = lax.iota(jnp.int32, L)

    def s1(b, carry):
        p0 = pu[b, pl.ds(0, L)]
        p1 = pu[b, pl.ds(L, L)]
        q0 = qi[b, pl.ds(0, L)]
        q1 = qi[b, pl.ds(L, L)]
        plsc.store_scatter(hbuf, [b * HPAD + lanes], p0 * q0 + p1 * q1)
        return carry

    lax.fori_loop(0, RPW, s1, 0)

    def s2(i, carry):
        rvec = i * L + lanes
        acc = plsc.load_gather(buv, [rvec]) + plsc.load_gather(biv, [rvec]) + MU
        hbase = rvec * HPAD
        for j in range(L):
            acc = acc + plsc.load_gather(hbuf, [hbase + j])
        plsc.store_scatter(outv, [rvec], acc)
        return carry

    lax.fori_loop(0, RPW // L, s2, 0)

    pltpu.sync_copy(outv, out_hbm.at[pl.ds(base, RPW)])


def kernel(x, P, Q, bu, bi):
    user = x[:, 0].astype(jnp.int32)
    item = x[:, 1].astype(jnp.int32)

    def swz(r):
        q = r % TBLK
        return (r // TBLK) * TBLK + 4 * (q % SLAB) + q // SLAB

    userT = swz(user).reshape(B // CHUNK, CHUNK)
    itemT = swz(item).reshape(B // CHUNK, CHUNK)
    userO = user.reshape(B // CHUNK, CHUNK)
    itemO = item.reshape(B // CHUNK, CHUNK)

    pb_flat, qb_flat, bu1, bi1 = pl.pallas_call(
        _tpose_body,
        grid=(TGRID,),
        in_specs=[
            pl.BlockSpec((F, TBLK), lambda i: (0, i)),
            pl.BlockSpec((F, TBLK), lambda i: (0, i)),
            pl.BlockSpec((1, TBLK), lambda i: (0, i)),
            pl.BlockSpec((1, TBLK), lambda i: (0, i)),
        ],
        out_specs=[
            pl.BlockSpec((TBLK * F,), lambda i: (i,)),
            pl.BlockSpec((TBLK * F,), lambda i: (i,)),
            pl.BlockSpec((TBLK,), lambda i: (i,)),
            pl.BlockSpec((TBLK,), lambda i: (i,)),
        ],
        out_shape=[
            jax.ShapeDtypeStruct((NPAD * F,), jnp.float32),
            jax.ShapeDtypeStruct((NPAD * F,), jnp.float32),
            jax.ShapeDtypeStruct((NPAD,), jnp.float32),
            jax.ShapeDtypeStruct((NPAD,), jnp.float32),
        ],
        compiler_params=pltpu.CompilerParams(
            fuse_transposed_lhs_in_matmul=True),
    )(P.T, Q.T, bu.T, bi.T)
    Pb = pb_flat.reshape(NPAD, F)
    Qb = qb_flat.reshape(NPAD, F)

    mesh = plsc.VectorSubcoreMesh(core_axis_name="c", subcore_axis_name="s")
    k = pl.kernel(
        _svdpp_body,
        out_type=jax.ShapeDtypeStruct((B,), jnp.float32),
        mesh=mesh,
        compiler_params=pltpu.CompilerParams(
            needs_layout_passes=False, use_tc_tiling_on_sc=False),
        scratch_types=[
            pltpu.VMEM((NCHUNK, CHUNK), jnp.int32),
            pltpu.VMEM((NCHUNK, CHUNK), jnp.int32),
            pltpu.VMEM((NCHUNK, CHUNK), jnp.int32),
            pltpu.VMEM((NCHUNK, CHUNK), jnp.int32),
            pltpu.VMEM((RPW, F), jnp.float32),
            pltpu.VMEM((RPW, F), jnp.float32),
            pltpu.VMEM((RPW,), jnp.float32),
            pltpu.VMEM((RPW,), jnp.float32),
            pltpu.VMEM((RPW * HPAD,), jnp.float32),
            pltpu.VMEM((RPW,), jnp.float32),
            pltpu.SemaphoreType.DMA,
        ],
    )
    return k(userT, itemT, userO, itemO, Pb, Qb, bu1, bi1)


# speedup vs baseline: 3.9313x; 1.2833x over previous
import jax
import jax.numpy as jnp
from jax import lax
from jax.experimental import pallas as pl
from jax.experimental.pallas import tpu as pltpu
from jax.experimental.pallas import tpu_sc as plsc

MU = 3.5
B = 16384
F = 32
N = 1000000
NC, NS, L = 2, 16, 16
NW = NC * NS
RPW = B // NW
CHUNK = 128
NCHUNK = RPW // CHUNK
HPAD = 17

TBLK = 8192
SLAB = TBLK // 4
TGRID = (N + TBLK - 1) // TBLK
NPAD = TGRID * TBLK


def _tpose_body(pt_ref, qt_ref, but_ref, bit_ref, po_ref, qo_ref,
                buo_ref, bio_ref):
    kk = lax.broadcasted_iota(jnp.int32, (F, 128), 0)
    jj = lax.broadcasted_iota(jnp.int32, (F, 128), 1)
    dn = (((0,), (0,)), ((), ()))
    for src, dst in ((pt_ref, po_ref), (qt_ref, qo_ref)):
        x = src[...].astype(jnp.bfloat16)
        y = None
        for a in range(4):
            sel = (jj == kk + 32 * a).astype(jnp.bfloat16)
            part = lax.dot_general(
                x[:, a * SLAB:(a + 1) * SLAB], sel, dn,
                preferred_element_type=jnp.float32)
            y = part if y is None else y + part
        dst[...] = y.reshape(TBLK * F)
    buo_ref[...] = but_ref[...].reshape(TBLK)
    bio_ref[...] = bit_ref[...].reshape(TBLK)


def _svdpp_body(ut_hbm, it_hbm, uo_hbm, io_hbm, P_hbm, Q_hbm,
                bu_hbm, bi_hbm, out_hbm,
                uidx, iidx, uorig, iorig, pu, qi, buv, biv, hbuf, outv, sem):
    wid = lax.axis_index("s") * NC + lax.axis_index("c")
    base = wid * RPW

    pltpu.sync_copy(ut_hbm.at[pl.ds(wid * NCHUNK, NCHUNK)], uidx)
    pltpu.sync_copy(it_hbm.at[pl.ds(wid * NCHUNK, NCHUNK)], iidx)
    pltpu.sync_copy(uo_hbm.at[pl.ds(wid * NCHUNK, NCHUNK)], uorig)
    pltpu.sync_copy(io_hbm.at[pl.ds(wid * NCHUNK, NCHUNK)], iorig)

    copies = []
    for j in range(NCHUNK):
        sl = pl.ds(j * CHUNK, CHUNK)
        copies.append(pltpu.async_copy(P_hbm.at[uidx.at[j]], pu.at[sl], sem))
        copies.append(pltpu.async_copy(Q_hbm.at[iidx.at[j]], qi.at[sl], sem))
        copies.append(pltpu.async_copy(bu_hbm.at[uorig.at[j]], buv.at[sl], sem))
        copies.append(pltpu.async_copy(bi_hbm.at[iorig.at[j]], biv.at[sl], sem))
    for c in copies:
        c.wait()

    lanes = lax.iota(jnp.int32, L)

    def s1(b, carry):
        p0 = pu[b, pl.ds(0, L)]
        p1 = pu[b, pl.ds(L, L)]
        q0 = qi[b, pl.ds(0, L)]
        q1 = qi[b, pl.ds(L, L)]
        plsc.store_scatter(hbuf, [b * HPAD + lanes], p0 * q0 + p1 * q1)
        return carry

    lax.fori_loop(0, RPW, s1, 0)

    def s2(i, carry):
        rvec = i * L + lanes
        acc = plsc.load_gather(buv, [rvec]) + plsc.load_gather(biv, [rvec]) + MU
        hbase = rvec * HPAD
        for j in range(L):
            acc = acc + plsc.load_gather(hbuf, [hbase + j])
        plsc.store_scatter(outv, [rvec], acc)
        return carry

    lax.fori_loop(0, RPW // L, s2, 0)

    pltpu.sync_copy(outv, out_hbm.at[pl.ds(base, RPW)])


def kernel(x, P, Q, bu, bi):
    user = x[:, 0].astype(jnp.int32)
    item = x[:, 1].astype(jnp.int32)

    def swz(r):
        q = r % TBLK
        return (r // TBLK) * TBLK + 4 * (q % SLAB) + q // SLAB

    userT = swz(user).reshape(B // CHUNK, CHUNK)
    itemT = swz(item).reshape(B // CHUNK, CHUNK)
    userO = user.reshape(B // CHUNK, CHUNK)
    itemO = item.reshape(B // CHUNK, CHUNK)

    pb_flat, qb_flat, bu1, bi1 = pl.pallas_call(
        _tpose_body,
        grid=(TGRID,),
        in_specs=[
            pl.BlockSpec((F, TBLK), lambda i: (0, i)),
            pl.BlockSpec((F, TBLK), lambda i: (0, i)),
            pl.BlockSpec((1, TBLK), lambda i: (0, i)),
            pl.BlockSpec((1, TBLK), lambda i: (0, i)),
        ],
        out_specs=[
            pl.BlockSpec((TBLK * F,), lambda i: (i,)),
            pl.BlockSpec((TBLK * F,), lambda i: (i,)),
            pl.BlockSpec((TBLK,), lambda i: (i,)),
            pl.BlockSpec((TBLK,), lambda i: (i,)),
        ],
        out_shape=[
            jax.ShapeDtypeStruct((NPAD * F,), jnp.float32),
            jax.ShapeDtypeStruct((NPAD * F,), jnp.float32),
            jax.ShapeDtypeStruct((NPAD,), jnp.float32),
            jax.ShapeDtypeStruct((NPAD,), jnp.float32),
        ],
        compiler_params=pltpu.CompilerParams(
            fuse_transposed_lhs_in_matmul=True),
    )(P.T, Q.T, bu.T, bi.T)
    Pb = pb_flat.reshape(NPAD, F)
    Qb = qb_flat.reshape(NPAD, F)

    mesh = plsc.VectorSubcoreMesh(core_axis_name="c", subcore_axis_name="s")
    k = pl.kernel(
        _svdpp_body,
        out_type=jax.ShapeDtypeStruct((B,), jnp.float32),
        mesh=mesh,
        compiler_params=pltpu.CompilerParams(
            needs_layout_passes=False, use_tc_tiling_on_sc=False),
        scratch_types=[
            pltpu.VMEM((NCHUNK, CHUNK), jnp.int32),
            pltpu.VMEM((NCHUNK, CHUNK), jnp.int32),
            pltpu.VMEM((NCHUNK, CHUNK), jnp.int32),
            pltpu.VMEM((NCHUNK, CHUNK), jnp.int32),
            pltpu.VMEM((RPW, F), jnp.float32),
            pltpu.VMEM((RPW, F), jnp.float32),
            pltpu.VMEM((RPW,), jnp.float32),
            pltpu.VMEM((RPW,), jnp.float32),
            pltpu.VMEM((RPW * HPAD,), jnp.float32),
            pltpu.VMEM((RPW,), jnp.float32),
            pltpu.SemaphoreType.DMA,
        ],
    )
    return k(userT, itemT, userO, itemO, Pb, Qb, bu1, bi1)


# speedup vs baseline: 4.6016x; 1.1705x over previous
import jax
import jax.numpy as jnp
from jax import lax
from jax.experimental import pallas as pl
from jax.experimental.pallas import tpu as pltpu
from jax.experimental.pallas import tpu_sc as plsc

MU = 3.5
B = 16384
F = 32
N = 1000000
NC, NS, L = 2, 16, 16
NW = NC * NS
RPW = B // NW
CHUNK = 128
NCHUNK = RPW // CHUNK
HPAD = 17

TBLK = 16384
SLAB = TBLK // 4
TGRID = (N + TBLK - 1) // TBLK
NPAD = TGRID * TBLK


def _tpose_body(pt_ref, qt_ref, but_ref, bit_ref, po_ref, qo_ref,
                buo_ref, bio_ref):
    kk = lax.broadcasted_iota(jnp.int32, (F, 128), 0)
    jj = lax.broadcasted_iota(jnp.int32, (F, 128), 1)
    dn = (((0,), (0,)), ((), ()))
    for src, dst in ((pt_ref, po_ref), (qt_ref, qo_ref)):
        x = src[...].astype(jnp.bfloat16)
        y = None
        for a in range(4):
            sel = (jj == kk + 32 * a).astype(jnp.bfloat16)
            part = lax.dot_general(
                x[:, a * SLAB:(a + 1) * SLAB], sel, dn,
                preferred_element_type=jnp.float32)
            y = part if y is None else y + part
        dst[...] = y.reshape(TBLK * F)
    buo_ref[...] = but_ref[...].reshape(TBLK)
    bio_ref[...] = bit_ref[...].reshape(TBLK)


def _svdpp_body(ut_hbm, it_hbm, uo_hbm, io_hbm, P_hbm, Q_hbm,
                bu_hbm, bi_hbm, out_hbm,
                uidx, iidx, uorig, iorig, pu, qi, buv, biv, hbuf, outv, sem):
    wid = lax.axis_index("s") * NC + lax.axis_index("c")
    base = wid * RPW

    pltpu.sync_copy(ut_hbm.at[pl.ds(wid * NCHUNK, NCHUNK)], uidx)
    pltpu.sync_copy(it_hbm.at[pl.ds(wid * NCHUNK, NCHUNK)], iidx)
    pltpu.sync_copy(uo_hbm.at[pl.ds(wid * NCHUNK, NCHUNK)], uorig)
    pltpu.sync_copy(io_hbm.at[pl.ds(wid * NCHUNK, NCHUNK)], iorig)

    copies = []
    for j in range(NCHUNK):
        sl = pl.ds(j * CHUNK, CHUNK)
        copies.append(pltpu.async_copy(P_hbm.at[uidx.at[j]], pu.at[sl], sem))
        copies.append(pltpu.async_copy(Q_hbm.at[iidx.at[j]], qi.at[sl], sem))
        copies.append(pltpu.async_copy(bu_hbm.at[uorig.at[j]], buv.at[sl], sem))
        copies.append(pltpu.async_copy(bi_hbm.at[iorig.at[j]], biv.at[sl], sem))
    for c in copies:
        c.wait()

    lanes = lax.iota(jnp.int32, L)

    def s1(b, carry):
        p0 = pu[b, pl.ds(0, L)]
        p1 = pu[b, pl.ds(L, L)]
        q0 = qi[b, pl.ds(0, L)]
        q1 = qi[b, pl.ds(L, L)]
        plsc.store_scatter(hbuf, [b * HPAD + lanes], p0 * q0 + p1 * q1)
        return carry

    lax.fori_loop(0, RPW, s1, 0)

    def s2(i, carry):
        rvec = i * L + lanes
        acc = plsc.load_gather(buv, [rvec]) + plsc.load_gather(biv, [rvec]) + MU
        hbase = rvec * HPAD
        for j in range(L):
            acc = acc + plsc.load_gather(hbuf, [hbase + j])
        plsc.store_scatter(outv, [rvec], acc)
        return carry

    lax.fori_loop(0, RPW // L, s2, 0)

    pltpu.sync_copy(outv, out_hbm.at[pl.ds(base, RPW)])


def kernel(x, P, Q, bu, bi):
    user = x[:, 0].astype(jnp.int32)
    item = x[:, 1].astype(jnp.int32)

    def swz(r):
        q = r % TBLK
        return (r // TBLK) * TBLK + 4 * (q % SLAB) + q // SLAB

    userT = swz(user).reshape(B // CHUNK, CHUNK)
    itemT = swz(item).reshape(B // CHUNK, CHUNK)
    userO = user.reshape(B // CHUNK, CHUNK)
    itemO = item.reshape(B // CHUNK, CHUNK)

    pb_flat, qb_flat, bu1, bi1 = pl.pallas_call(
        _tpose_body,
        grid=(TGRID,),
        in_specs=[
            pl.BlockSpec((F, TBLK), lambda i: (0, i)),
            pl.BlockSpec((F, TBLK), lambda i: (0, i)),
            pl.BlockSpec((1, TBLK), lambda i: (0, i)),
            pl.BlockSpec((1, TBLK), lambda i: (0, i)),
        ],
        out_specs=[
            pl.BlockSpec((TBLK * F,), lambda i: (i,)),
            pl.BlockSpec((TBLK * F,), lambda i: (i,)),
            pl.BlockSpec((TBLK,), lambda i: (i,)),
            pl.BlockSpec((TBLK,), lambda i: (i,)),
        ],
        out_shape=[
            jax.ShapeDtypeStruct((NPAD * F,), jnp.float32),
            jax.ShapeDtypeStruct((NPAD * F,), jnp.float32),
            jax.ShapeDtypeStruct((NPAD,), jnp.float32),
            jax.ShapeDtypeStruct((NPAD,), jnp.float32),
        ],
        compiler_params=pltpu.CompilerParams(
            fuse_transposed_lhs_in_matmul=True),
    )(P.T, Q.T, bu.T, bi.T)
    Pb = pb_flat.reshape(NPAD, F)
    Qb = qb_flat.reshape(NPAD, F)

    mesh = plsc.VectorSubcoreMesh(core_axis_name="c", subcore_axis_name="s")
    k = pl.kernel(
        _svdpp_body,
        out_type=jax.ShapeDtypeStruct((B,), jnp.float32),
        mesh=mesh,
        compiler_params=pltpu.CompilerParams(
            needs_layout_passes=False, use_tc_tiling_on_sc=False),
        scratch_types=[
            pltpu.VMEM((NCHUNK, CHUNK), jnp.int32),
            pltpu.VMEM((NCHUNK, CHUNK), jnp.int32),
            pltpu.VMEM((NCHUNK, CHUNK), jnp.int32),
            pltpu.VMEM((NCHUNK, CHUNK), jnp.int32),
            pltpu.VMEM((RPW, F), jnp.float32),
            pltpu.VMEM((RPW, F), jnp.float32),
            pltpu.VMEM((RPW,), jnp.float32),
            pltpu.VMEM((RPW,), jnp.float32),
            pltpu.VMEM((RPW * HPAD,), jnp.float32),
            pltpu.VMEM((RPW,), jnp.float32),
            pltpu.SemaphoreType.DMA,
        ],
    )
    return k(userT, itemT, userO, itemO, Pb, Qb, bu1, bi1)


# speedup vs baseline: 5.0300x; 1.0931x over previous
import jax
import jax.numpy as jnp
from jax import lax
from jax.experimental import pallas as pl
from jax.experimental.pallas import tpu as pltpu
from jax.experimental.pallas import tpu_sc as plsc

MU = 3.5
B = 16384
F = 32
N = 1000000
NC, NS, L = 2, 16, 16
NW = NC * NS
RPW = B // NW
CHUNK = 128
NCHUNK = RPW // CHUNK
HPAD = 17

TBLK = 32768
SLAB = TBLK // 4
TGRID = (N + TBLK - 1) // TBLK
NPAD = TGRID * TBLK


def _tpose_body(pt_ref, qt_ref, but_ref, bit_ref, po_ref, qo_ref,
                buo_ref, bio_ref):
    kk = lax.broadcasted_iota(jnp.int32, (F, 128), 0)
    jj = lax.broadcasted_iota(jnp.int32, (F, 128), 1)
    dn = (((0,), (0,)), ((), ()))
    for src, dst in ((pt_ref, po_ref), (qt_ref, qo_ref)):
        x = src[...].astype(jnp.bfloat16)
        y = None
        for a in range(4):
            sel = (jj == kk + 32 * a).astype(jnp.bfloat16)
            part = lax.dot_general(
                x[:, a * SLAB:(a + 1) * SLAB], sel, dn,
                preferred_element_type=jnp.float32)
            y = part if y is None else y + part
        dst[...] = y.reshape(TBLK * F)
    buo_ref[...] = but_ref[...].reshape(TBLK)
    bio_ref[...] = bit_ref[...].reshape(TBLK)


def _svdpp_body(ut_hbm, it_hbm, uo_hbm, io_hbm, P_hbm, Q_hbm,
                bu_hbm, bi_hbm, out_hbm,
                uidx, iidx, uorig, iorig, pu, qi, buv, biv, hbuf, outv, sem):
    wid = lax.axis_index("s") * NC + lax.axis_index("c")
    base = wid * RPW

    pltpu.sync_copy(ut_hbm.at[pl.ds(wid * NCHUNK, NCHUNK)], uidx)
    pltpu.sync_copy(it_hbm.at[pl.ds(wid * NCHUNK, NCHUNK)], iidx)
    pltpu.sync_copy(uo_hbm.at[pl.ds(wid * NCHUNK, NCHUNK)], uorig)
    pltpu.sync_copy(io_hbm.at[pl.ds(wid * NCHUNK, NCHUNK)], iorig)

    copies = []
    for j in range(NCHUNK):
        sl = pl.ds(j * CHUNK, CHUNK)
        copies.append(pltpu.async_copy(P_hbm.at[uidx.at[j]], pu.at[sl], sem))
        copies.append(pltpu.async_copy(Q_hbm.at[iidx.at[j]], qi.at[sl], sem))
        copies.append(pltpu.async_copy(bu_hbm.at[uorig.at[j]], buv.at[sl], sem))
        copies.append(pltpu.async_copy(bi_hbm.at[iorig.at[j]], biv.at[sl], sem))
    for c in copies:
        c.wait()

    lanes = lax.iota(jnp.int32, L)

    def s1(b, carry):
        p0 = pu[b, pl.ds(0, L)]
        p1 = pu[b, pl.ds(L, L)]
        q0 = qi[b, pl.ds(0, L)]
        q1 = qi[b, pl.ds(L, L)]
        plsc.store_scatter(hbuf, [b * HPAD + lanes], p0 * q0 + p1 * q1)
        return carry

    lax.fori_loop(0, RPW, s1, 0)

    def s2(i, carry):
        rvec = i * L + lanes
        acc = plsc.load_gather(buv, [rvec]) + plsc.load_gather(biv, [rvec]) + MU
        hbase = rvec * HPAD
        for j in range(L):
            acc = acc + plsc.load_gather(hbuf, [hbase + j])
        plsc.store_scatter(outv, [rvec], acc)
        return carry

    lax.fori_loop(0, RPW // L, s2, 0)

    pltpu.sync_copy(outv, out_hbm.at[pl.ds(base, RPW)])


def kernel(x, P, Q, bu, bi):
    user = x[:, 0].astype(jnp.int32)
    item = x[:, 1].astype(jnp.int32)

    def swz(r):
        q = r % TBLK
        return (r // TBLK) * TBLK + 4 * (q % SLAB) + q // SLAB

    userT = swz(user).reshape(B // CHUNK, CHUNK)
    itemT = swz(item).reshape(B // CHUNK, CHUNK)
    userO = user.reshape(B // CHUNK, CHUNK)
    itemO = item.reshape(B // CHUNK, CHUNK)

    pb_flat, qb_flat, bu1, bi1 = pl.pallas_call(
        _tpose_body,
        grid=(TGRID,),
        in_specs=[
            pl.BlockSpec((F, TBLK), lambda i: (0, i)),
            pl.BlockSpec((F, TBLK), lambda i: (0, i)),
            pl.BlockSpec((1, TBLK), lambda i: (0, i)),
            pl.BlockSpec((1, TBLK), lambda i: (0, i)),
        ],
        out_specs=[
            pl.BlockSpec((TBLK * F,), lambda i: (i,)),
            pl.BlockSpec((TBLK * F,), lambda i: (i,)),
            pl.BlockSpec((TBLK,), lambda i: (i,)),
            pl.BlockSpec((TBLK,), lambda i: (i,)),
        ],
        out_shape=[
            jax.ShapeDtypeStruct((NPAD * F,), jnp.float32),
            jax.ShapeDtypeStruct((NPAD * F,), jnp.float32),
            jax.ShapeDtypeStruct((NPAD,), jnp.float32),
            jax.ShapeDtypeStruct((NPAD,), jnp.float32),
        ],
        compiler_params=pltpu.CompilerParams(
            fuse_transposed_lhs_in_matmul=True),
    )(P.T, Q.T, bu.T, bi.T)
    Pb = pb_flat.reshape(NPAD, F)
    Qb = qb_flat.reshape(NPAD, F)

    mesh = plsc.VectorSubcoreMesh(core_axis_name="c", subcore_axis_name="s")
    k = pl.kernel(
        _svdpp_body,
        out_type=jax.ShapeDtypeStruct((B,), jnp.float32),
        mesh=mesh,
        compiler_params=pltpu.CompilerParams(
            needs_layout_passes=False, use_tc_tiling_on_sc=False),
        scratch_types=[
            pltpu.VMEM((NCHUNK, CHUNK), jnp.int32),
            pltpu.VMEM((NCHUNK, CHUNK), jnp.int32),
            pltpu.VMEM((NCHUNK, CHUNK), jnp.int32),
            pltpu.VMEM((NCHUNK, CHUNK), jnp.int32),
            pltpu.VMEM((RPW, F), jnp.float32),
            pltpu.VMEM((RPW, F), jnp.float32),
            pltpu.VMEM((RPW,), jnp.float32),
            pltpu.VMEM((RPW,), jnp.float32),
            pltpu.VMEM((RPW * HPAD,), jnp.float32),
            pltpu.VMEM((RPW,), jnp.float32),
            pltpu.SemaphoreType.DMA,
        ],
    )
    return k(userT, itemT, userO, itemO, Pb, Qb, bu1, bi1)
